# baseline (speedup 1.0000x reference)
import functools

import jax
import jax.numpy as jnp
from jax import lax
from jax.experimental import pallas as pl
from jax.experimental.pallas import tpu as pltpu


def _fused_kernel(x_hbm, batch_hbm, glob_ref, w_ref, b_ref, out_ref,
                  x_buf, b_buf, acc_ref, x_sem, b_sem,
                  *, chunk, nsteps, nbuf):
    n_graphs = acc_ref.shape[0]
    h = glob_ref.shape[1]

    def start_in(i):
        slot = lax.rem(i, nbuf)
        pltpu.make_async_copy(x_hbm.at[pl.ds(i * chunk, chunk), :],
                              x_buf.at[slot], x_sem.at[slot]).start()
        pltpu.make_async_copy(batch_hbm.at[:, pl.ds(i * chunk, chunk)],
                              b_buf.at[slot], b_sem.at[slot]).start()

    def wait_in(slot):
        pltpu.make_async_copy(x_hbm.at[pl.ds(0, chunk), :],
                              x_buf.at[slot], x_sem.at[slot]).wait()
        pltpu.make_async_copy(batch_hbm.at[:, pl.ds(0, chunk)],
                              b_buf.at[slot], b_sem.at[slot]).wait()

    for i in range(min(nbuf - 1, nsteps)):
        start_in(i)

    acc_ref[...] = jnp.zeros_like(acc_ref)
    graph_iota = lax.broadcasted_iota(jnp.int32, (n_graphs, chunk), 0)

    def body(i, _):
        @pl.when(i + nbuf - 1 < nsteps)
        def _prefetch():
            start_in(i + nbuf - 1)
        slot = lax.rem(i, nbuf)
        wait_in(slot)
        mask = (b_buf[slot] == graph_iota).astype(jnp.bfloat16)
        acc_ref[...] += jnp.dot(mask, x_buf[slot].astype(jnp.bfloat16),
                                preferred_element_type=jnp.float32)
        return ()

    lax.fori_loop(0, nsteps, body, (), unroll=False)

    w = w_ref[...]
    dn = (((1,), (1,)), ((), ()))
    out = (lax.dot_general(glob_ref[...], w[:, :h], dn,
                           preferred_element_type=jnp.float32)
           + lax.dot_general(acc_ref[...], w[:, h:], dn,
                             preferred_element_type=jnp.float32)
           + b_ref[...])
    out_ref[...] = out.astype(out_ref.dtype)


def kernel(x, glob, batch, weight, bias):
    n_nodes, h = x.shape
    b_graphs = glob.shape[0]
    out_dtype = jnp.result_type(x.dtype, glob.dtype, weight.dtype)

    chunk = 8192
    while n_nodes % chunk:
        chunk //= 2
    nsteps = n_nodes // chunk
    nbuf = min(3, nsteps)

    batch2d = batch.astype(jnp.int32).reshape(1, n_nodes)
    bias2d = bias.reshape(1, h)

    out = pl.pallas_call(
        functools.partial(_fused_kernel, chunk=chunk, nsteps=nsteps,
                          nbuf=nbuf),
        out_shape=jax.ShapeDtypeStruct((b_graphs, h), out_dtype),
        grid=(1,),
        in_specs=[
            pl.BlockSpec(memory_space=pl.ANY),
            pl.BlockSpec(memory_space=pl.ANY),
            pl.BlockSpec((b_graphs, h), lambda n: (0, 0)),
            pl.BlockSpec((h, 2 * h), lambda n: (0, 0)),
            pl.BlockSpec((1, h), lambda n: (0, 0)),
        ],
        out_specs=pl.BlockSpec((b_graphs, h), lambda n: (0, 0)),
        scratch_shapes=[
            pltpu.VMEM((nbuf, chunk, h), x.dtype),
            pltpu.VMEM((nbuf, 1, chunk), jnp.int32),
            pltpu.VMEM((b_graphs, h), jnp.float32),
            pltpu.SemaphoreType.DMA((nbuf,)),
            pltpu.SemaphoreType.DMA((nbuf,)),
        ],
        compiler_params=pltpu.CompilerParams(
            dimension_semantics=("arbitrary",),
        ),
        cost_estimate=pl.CostEstimate(
            flops=2 * b_graphs * n_nodes * h + 4 * b_graphs * h * h,
            transcendentals=0,
            bytes_accessed=n_nodes * h * x.dtype.itemsize + n_nodes * 4
                           + 2 * h * h * weight.dtype.itemsize
                           + 2 * b_graphs * h * 4,
        ),
    )(x, batch2d, glob, weight, bias2d)

    return out


# speedup vs baseline: 1.0736x; 1.0736x over previous
import functools

import jax
import jax.numpy as jnp
from jax import lax
from jax.experimental import pallas as pl
from jax.experimental.pallas import tpu as pltpu


def _fused_kernel(x_ref, batch_ref, glob_ref, w_ref, b_ref, out_ref,
                  acc_ref, gw_ref, *, tile_n, n_nodes):
    n = pl.program_id(0)
    n_graphs = acc_ref.shape[0]
    h = x_ref.shape[1]
    dn = (((1,), (1,)), ((), ()))

    @pl.when(n == 0)
    def _init():
        acc_ref[...] = jnp.zeros_like(acc_ref)
        gw_ref[...] = lax.dot_general(glob_ref[...], w_ref[...][:, :h], dn,
                                      preferred_element_type=jnp.float32
                                      ) + b_ref[...]

    x_t = x_ref[...]
    if n_nodes % tile_n != 0:
        row_ids = n * tile_n + lax.broadcasted_iota(jnp.int32, (tile_n, 1), 0)
        x_t = jnp.where(row_ids < n_nodes, x_t, 0)

    ids = batch_ref[...]
    graph_iota = lax.broadcasted_iota(jnp.int32, (n_graphs, tile_n), 0)
    mask = (ids == graph_iota).astype(jnp.bfloat16)

    acc_ref[...] += jnp.dot(mask, x_t.astype(jnp.bfloat16),
                            preferred_element_type=jnp.float32)

    @pl.when(n == pl.num_programs(0) - 1)
    def _finalize():
        out = (lax.dot_general(acc_ref[...], w_ref[...][:, h:], dn,
                               preferred_element_type=jnp.float32)
               + gw_ref[...])
        out_ref[...] = out.astype(out_ref.dtype)


def kernel(x, glob, batch, weight, bias):
    n_nodes, h = x.shape
    b_graphs = glob.shape[0]
    out_dtype = jnp.result_type(x.dtype, glob.dtype, weight.dtype)

    tile_n = min(8192, max(128, ((n_nodes + 127) // 128) * 128))
    steps = pl.cdiv(n_nodes, tile_n)

    batch2d = batch.astype(jnp.int32).reshape(1, n_nodes)
    bias2d = bias.reshape(1, h)

    out = pl.pallas_call(
        functools.partial(_fused_kernel, tile_n=tile_n, n_nodes=n_nodes),
        out_shape=jax.ShapeDtypeStruct((b_graphs, h), out_dtype),
        grid=(steps,),
        in_specs=[
            pl.BlockSpec((tile_n, h), lambda n: (n, 0)),
            pl.BlockSpec((1, tile_n), lambda n: (0, n)),
            pl.BlockSpec((b_graphs, h), lambda n: (0, 0)),
            pl.BlockSpec((h, 2 * h), lambda n: (0, 0)),
            pl.BlockSpec((1, h), lambda n: (0, 0)),
        ],
        out_specs=pl.BlockSpec((b_graphs, h), lambda n: (0, 0)),
        scratch_shapes=[pltpu.VMEM((b_graphs, h), jnp.float32),
                        pltpu.VMEM((b_graphs, h), jnp.float32)],
        compiler_params=pltpu.CompilerParams(
            dimension_semantics=("arbitrary",),
        ),
        cost_estimate=pl.CostEstimate(
            flops=2 * b_graphs * n_nodes * h + 4 * b_graphs * h * h,
            transcendentals=0,
            bytes_accessed=n_nodes * h * x.dtype.itemsize + n_nodes * 4
                           + 2 * h * h * weight.dtype.itemsize
                           + 2 * b_graphs * h * 4,
        ),
    )(x, batch2d, glob, weight, bias2d)

    return out


# speedup vs baseline: 1.0826x; 1.0084x over previous
import functools

import jax
import jax.numpy as jnp
from jax import lax
from jax.experimental import pallas as pl
from jax.experimental.pallas import tpu as pltpu


def _fused_kernel(x_ref, batch_ref, glob_ref, w_ref, b_ref, out_ref,
                  acc_ref, *, tile_n, n_nodes):
    n = pl.program_id(0)
    n_graphs = acc_ref.shape[0]
    h = x_ref.shape[1]

    @pl.when(n == 0)
    def _init():
        acc_ref[...] = jnp.zeros_like(acc_ref)

    x_t = x_ref[...]
    if n_nodes % tile_n != 0:
        row_ids = n * tile_n + lax.broadcasted_iota(jnp.int32, (tile_n, 1), 0)
        x_t = jnp.where(row_ids < n_nodes, x_t, 0)

    ids = batch_ref[...]
    graph_iota = lax.broadcasted_iota(jnp.int32, (n_graphs, tile_n), 0)
    mask = (ids == graph_iota).astype(jnp.bfloat16)

    acc_ref[...] += jnp.dot(mask, x_t.astype(jnp.bfloat16),
                            preferred_element_type=jnp.float32)

    @pl.when(n == pl.num_programs(0) - 1)
    def _finalize():
        w = w_ref[...]
        dn = (((1,), (1,)), ((), ()))
        out = (lax.dot_general(glob_ref[...], w[:, :h], dn,
                               preferred_element_type=jnp.float32)
               + lax.dot_general(acc_ref[...], w[:, h:], dn,
                                 preferred_element_type=jnp.float32)
               + b_ref[...])
        out_ref[...] = out.astype(out_ref.dtype)


def kernel(x, glob, batch, weight, bias):
    n_nodes, h = x.shape
    b_graphs = glob.shape[0]
    out_dtype = jnp.result_type(x.dtype, glob.dtype, weight.dtype)

    tile_n = min(8192, max(128, ((n_nodes + 127) // 128) * 128))
    steps = pl.cdiv(n_nodes, tile_n)

    batch2d = batch.astype(jnp.int32).reshape(1, n_nodes)
    bias2d = bias.reshape(1, h)

    out = pl.pallas_call(
        functools.partial(_fused_kernel, tile_n=tile_n, n_nodes=n_nodes),
        out_shape=jax.ShapeDtypeStruct((b_graphs, h), out_dtype),
        grid=(steps,),
        in_specs=[
            pl.BlockSpec((tile_n, h), lambda n: (n, 0)),
            pl.BlockSpec((1, tile_n), lambda n: (0, n)),
            pl.BlockSpec((b_graphs, h), lambda n: (0, 0)),
            pl.BlockSpec((h, 2 * h), lambda n: (0, 0)),
            pl.BlockSpec((1, h), lambda n: (0, 0)),
        ],
        out_specs=pl.BlockSpec((b_graphs, h), lambda n: (0, 0)),
        scratch_shapes=[pltpu.VMEM((b_graphs, h), jnp.float32)],
        compiler_params=pltpu.CompilerParams(
            dimension_semantics=("arbitrary",),
        ),
        cost_estimate=pl.CostEstimate(
            flops=2 * b_graphs * n_nodes * h + 4 * b_graphs * h * h,
            transcendentals=0,
            bytes_accessed=n_nodes * h * x.dtype.itemsize + n_nodes * 4
                           + 2 * h * h * weight.dtype.itemsize
                           + 2 * b_graphs * h * 4,
        ),
    )(x, batch2d, glob, weight, bias2d)

    return out
